# baseline (speedup 1.0000x reference)
import functools

import jax
import jax.numpy as jnp
from jax import lax
from jax.experimental import pallas as pl
from jax.experimental.pallas import tpu as pltpu
from jax.experimental.pallas import tpu_sc as plsc

N = 100000
E = 3200000
NUM_GRAPHS = 512
HIDDEN = 64

NP = 100352
EP = 3211264
RPS = NP // 16
ZROWS = 98
CHUNK = 512
PCH = CHUNK // 128
OUTER = (EP // 16) // CHUNK
OUTER_DEG = (EP // 32) // CHUNK
BLK = 2048
GRID = NP // BLK

_HI = lax.Precision.HIGHEST


def _mesh():
    return plsc.VectorSubcoreMesh(core_axis_name="c", subcore_axis_name="s")


_SC_PARAMS = pltpu.CompilerParams(use_tc_tiling_on_sc=False)


def _deg_body(dst_hbm, ones_hbm, zeros_hbm, out0, out1, dst_v, ones_v, deg_sh, ssem):
    c = lax.axis_index("c")
    s = lax.axis_index("s")
    pltpu.sync_copy(ones_hbm, ones_v)
    pltpu.sync_copy(zeros_hbm, deg_sh.at[pl.ds(s * RPS, RPS)])
    plsc.subcore_barrier()

    base128 = (c * 16 + s) * ((EP // 32) // 128)

    def drain_scatters(b):
        for j in range(PCH):
            pltpu.make_async_copy(ones_v, deg_sh.at[dst_v.at[b * PCH + j]],
                                  ssem).wait()

    def outer(i, carry):
        b = jnp.bitwise_and(i, 1)

        @pl.when(i >= 2)
        def _():
            drain_scatters(b)

        @pl.when(i < OUTER_DEG)
        def _():
            off128 = base128 + i * PCH
            pltpu.sync_copy(dst_hbm.at[pl.ds(off128, PCH)],
                            dst_v.at[pl.ds(b * PCH, PCH)])
            for j in range(PCH):
                pltpu.async_copy(ones_v, deg_sh.at[dst_v.at[b * PCH + j]],
                                 ssem, add=True)
        return carry

    lax.fori_loop(0, OUTER_DEG + 2, outer, 0)
    plsc.subcore_barrier()
    r = pl.ds(s * RPS, RPS)

    @pl.when(c == 0)
    def _():
        pltpu.sync_copy(deg_sh.at[r], out0.at[r])

    @pl.when(c == 1)
    def _():
        pltpu.sync_copy(deg_sh.at[r], out1.at[r])


def _deg_call(dst2, ones_deg, zeros_deg):
    return pl.kernel(
        _deg_body,
        out_type=(jax.ShapeDtypeStruct((NP, 1), jnp.float32),
                  jax.ShapeDtypeStruct((NP, 1), jnp.float32)),
        mesh=_mesh(),
        scratch_types=[
            pltpu.VMEM((2 * PCH, 128), jnp.int32),
            pltpu.VMEM((128, 1), jnp.float32),
            pltpu.VMEM_SHARED((NP, 1), jnp.float32),
            pltpu.SemaphoreType.DMA,
        ],
        compiler_params=_SC_PARAMS,
    )(dst2, ones_deg, zeros_deg)


def _agg_body(hs0, hs1, hs2, hs3, ei_hbm, zeros_hbm,
              out0, out1, out2, out3,
              idx_v, rows_v, zbuf, agg_sh, gsem, ssem, isem):
    c = lax.axis_index("c")
    s = lax.axis_index("s")
    pltpu.sync_copy(zeros_hbm, zbuf)

    def do_quarter(hs_hbm, out_hbm):
        row0 = s * RPS
        for z in range(RPS // ZROWS):
            pltpu.sync_copy(zbuf, agg_sh.at[pl.ds(row0 + z * ZROWS, ZROWS)])
        plsc.subcore_barrier()

        base128 = s * ((EP // 16) // 128)

        def fire_idx(i):
            pltpu.async_copy(ei_hbm.at[pl.ds(base128 + i * PCH, PCH)],
                             idx_v.at[lax.rem(i, 4)], isem)

        def wait_idx(i):
            pltpu.make_async_copy(ei_hbm.at[pl.ds(base128 + i * PCH, PCH)],
                                  idx_v.at[lax.rem(i, 4)], isem).wait()

        fire_idx(0)

        def outer(i, carry):
            r3 = lax.rem(i, 3)
            r4 = lax.rem(i, 4)

            @pl.when(jnp.logical_and(i >= 3, i <= OUTER + 2))
            def _():
                p4 = lax.rem(i - 3, 4)
                for j in range(PCH):
                    pltpu.make_async_copy(
                        rows_v.at[pl.ds(r3 * CHUNK + 128 * j, 128)],
                        agg_sh.at[idx_v.at[p4, j, 1]], ssem).wait()

            @pl.when(i < OUTER)
            def _():
                wait_idx(i)

                @pl.when(i + 1 < OUTER)
                def _():
                    fire_idx(i + 1)

                for j in range(PCH):
                    pltpu.async_copy(hs_hbm.at[idx_v.at[r4, j, 0]],
                                     rows_v.at[pl.ds(r3 * CHUNK + 128 * j, 128)],
                                     gsem)

            @pl.when(jnp.logical_and(i >= 1, i <= OUTER))
            def _():
                p3 = lax.rem(i - 1, 3)
                p4 = lax.rem(i - 1, 4)
                for j in range(PCH):
                    pltpu.make_async_copy(
                        hs_hbm.at[idx_v.at[p4, j, 0]],
                        rows_v.at[pl.ds(p3 * CHUNK + 128 * j, 128)],
                        gsem).wait()
                for j in range(PCH):
                    pltpu.async_copy(
                        rows_v.at[pl.ds(p3 * CHUNK + 128 * j, 128)],
                        agg_sh.at[idx_v.at[p4, j, 1]], ssem, add=True)
            return carry

        lax.fori_loop(0, OUTER + 3, outer, 0)
        plsc.subcore_barrier()
        for z in range(RPS // ZROWS):
            r = pl.ds(row0 + z * ZROWS, ZROWS)
            pltpu.sync_copy(agg_sh.at[r], out_hbm.at[r])
        plsc.subcore_barrier()

    @pl.when(c == 0)
    def _():
        do_quarter(hs0, out0)
        do_quarter(hs1, out1)

    @pl.when(c == 1)
    def _():
        do_quarter(hs2, out2)
        do_quarter(hs3, out3)


def _agg_call(hsq, ei2, zeros_agg):
    q16 = jax.ShapeDtypeStruct((NP, 16), jnp.float32)
    return pl.kernel(
        _agg_body,
        out_type=(q16, q16, q16, q16),
        mesh=_mesh(),
        scratch_types=[
            pltpu.VMEM((4, PCH, 2, 128), jnp.int32),
            pltpu.VMEM((3 * CHUNK, 16), jnp.float32),
            pltpu.VMEM((ZROWS, 16), jnp.float32),
            pltpu.VMEM_SHARED((NP, 16), jnp.float32),
            pltpu.SemaphoreType.DMA,
            pltpu.SemaphoreType.DMA,
            pltpu.SemaphoreType.DMA,
        ],
        compiler_params=_SC_PARAMS,
    )(hsq[0], hsq[1], hsq[2], hsq[3], ei2, zeros_agg)


_QSPEC = pl.BlockSpec((BLK, 16), lambda i: (i, 0))
_CSPEC = pl.BlockSpec((BLK, 1), lambda i: (i, 0))


def _k1_body(deg0_ref, deg1_ref, x_ref, w_ref,
             o0, o1, o2, o3, dis_ref):
    dis = lax.rsqrt(deg0_ref[...] + deg1_ref[...] + 1.0)
    h = jnp.dot(x_ref[...], w_ref[...], precision=_HI)
    for q, o in enumerate((o0, o1, o2, o3)):
        o[...] = dis * h[:, 16 * q:16 * (q + 1)]
    dis_ref[...] = dis


def _k1_call(deg0, deg1, xP, W1):
    q16 = jax.ShapeDtypeStruct((NP, 16), jnp.float32)
    outs = pl.pallas_call(
        _k1_body,
        grid=(GRID,),
        in_specs=[
            _CSPEC,
            _CSPEC,
            pl.BlockSpec((BLK, xP.shape[1]), lambda i: (i, 0)),
            pl.BlockSpec(W1.shape, lambda i: (0, 0)),
        ],
        out_specs=[_QSPEC, _QSPEC, _QSPEC, _QSPEC, _CSPEC],
        out_shape=[q16, q16, q16, q16,
                   jax.ShapeDtypeStruct((NP, 1), jnp.float32)],
    )(deg0, deg1, xP, W1)
    return outs[:4], outs[4]


def _k2_body(a0, a1, a2, a3, h0, h1, h2, h3, dis_ref, b_ref, w_ref,
             o0, o1, o2, o3):
    dis = dis_ref[...]
    aq = (a0, a1, a2, a3)
    hq = (h0, h1, h2, h3)
    h = None
    for q in range(4):
        xn = jnp.maximum(dis * (aq[q][...] + hq[q][...])
                         + b_ref[:, 16 * q:16 * (q + 1)], 0.0)
        p = jnp.dot(xn, w_ref[16 * q:16 * (q + 1), :], precision=_HI)
        h = p if h is None else h + p
    for q, o in enumerate((o0, o1, o2, o3)):
        o[...] = dis * h[:, 16 * q:16 * (q + 1)]


def _k2_call(aggq, hsq, dis, b_row, W):
    q16 = jax.ShapeDtypeStruct((NP, 16), jnp.float32)
    return pl.pallas_call(
        _k2_body,
        grid=(GRID,),
        in_specs=[
            _QSPEC, _QSPEC, _QSPEC, _QSPEC,
            _QSPEC, _QSPEC, _QSPEC, _QSPEC,
            _CSPEC,
            pl.BlockSpec((1, HIDDEN), lambda i: (0, 0)),
            pl.BlockSpec((HIDDEN, HIDDEN), lambda i: (0, 0)),
        ],
        out_specs=[_QSPEC, _QSPEC, _QSPEC, _QSPEC],
        out_shape=[q16, q16, q16, q16],
    )(*aggq, *hsq, dis, b_row, W)


def _k4_body(a0, a1, a2, a3, h0, h1, h2, h3, dis_ref, b_ref, batch_ref,
             wl_ref, bl_ref, out_ref, segsum, cnt):
    i = pl.program_id(0)

    @pl.when(i == 0)
    def _():
        segsum[...] = jnp.zeros_like(segsum)
        cnt[...] = jnp.zeros_like(cnt)

    dis = dis_ref[...]
    iota = lax.broadcasted_iota(jnp.int32, (BLK, NUM_GRAPHS), 1)
    onehot = (batch_ref[...] == iota).astype(jnp.float32)
    aq = (a0, a1, a2, a3)
    hq = (h0, h1, h2, h3)
    for q in range(4):
        x3q = dis * (aq[q][...] + hq[q][...]) + b_ref[:, 16 * q:16 * (q + 1)]
        segsum[:, 16 * q:16 * (q + 1)] += lax.dot_general(
            onehot, x3q, (((0,), (0,)), ((), ())), precision=_HI)
    cnt[...] += lax.dot_general(onehot, jnp.ones((BLK, 1), jnp.float32),
                                (((0,), (0,)), ((), ())), precision=_HI)

    @pl.when(i == GRID - 1)
    def _():
        pooled = segsum[...] / jnp.maximum(cnt[...], 1.0)
        out_ref[...] = jnp.dot(pooled, wl_ref[...], precision=_HI) + bl_ref[...]


def _k4_call(aggq, hsq, dis, b_row, batchP, Wl, bl_row):
    return pl.pallas_call(
        _k4_body,
        grid=(GRID,),
        in_specs=[
            _QSPEC, _QSPEC, _QSPEC, _QSPEC,
            _QSPEC, _QSPEC, _QSPEC, _QSPEC,
            _CSPEC,
            pl.BlockSpec((1, HIDDEN), lambda i: (0, 0)),
            pl.BlockSpec((BLK, 1), lambda i: (i, 0)),
            pl.BlockSpec(Wl.shape, lambda i: (0, 0)),
            pl.BlockSpec((1, Wl.shape[1]), lambda i: (0, 0)),
        ],
        out_specs=pl.BlockSpec((NUM_GRAPHS, Wl.shape[1]), lambda i: (0, 0)),
        out_shape=jax.ShapeDtypeStruct((NUM_GRAPHS, Wl.shape[1]), jnp.float32),
        scratch_shapes=[
            pltpu.VMEM((NUM_GRAPHS, HIDDEN), jnp.float32),
            pltpu.VMEM((NUM_GRAPHS, 1), jnp.float32),
        ],
    )(*aggq, *hsq, dis, b_row, batchP, Wl, bl_row)


def kernel(x, edge_index, batch, W1, b1, W2, b2, W3, b3, Wl, bl):
    f32 = jnp.float32
    src, dst = edge_index[0], edge_index[1]
    npad = EP - E
    pad_idx = (N + (jnp.arange(npad, dtype=jnp.int32) % (NP - N))).astype(
        jnp.int32)
    src2 = jnp.concatenate([src, pad_idx]).reshape(EP // 128, 128)
    dst2 = jnp.concatenate([dst, pad_idx]).reshape(EP // 128, 128)
    ei2 = jnp.stack([src2, dst2], axis=1)
    xP = jnp.pad(x, ((0, NP - N), (0, 0)))
    batchP = jnp.pad(batch, (0, NP - N),
                     constant_values=NUM_GRAPHS).reshape(NP, 1)
    zeros_agg = jnp.zeros((ZROWS, 16), f32)
    zeros_deg = jnp.zeros((RPS, 1), f32)
    ones_deg = jnp.ones((128, 1), f32)

    deg0, deg1 = _deg_call(dst2, ones_deg, zeros_deg)
    hs1q, dis = _k1_call(deg0, deg1, xP, W1)
    agg1q = _agg_call(hs1q, ei2, zeros_agg)
    hs2q = _k2_call(agg1q, hs1q, dis, b1[None, :], W2)
    agg2q = _agg_call(hs2q, ei2, zeros_agg)
    hs3q = _k2_call(agg2q, hs2q, dis, b2[None, :], W3)
    agg3q = _agg_call(hs3q, ei2, zeros_agg)
    return _k4_call(agg3q, hs3q, dis, b3[None, :], batchP, Wl, bl[None, :])


# speedup vs baseline: 1.1559x; 1.1559x over previous
import functools

import jax
import jax.numpy as jnp
from jax import lax
from jax.experimental import pallas as pl
from jax.experimental.pallas import tpu as pltpu
from jax.experimental.pallas import tpu_sc as plsc

N = 100000
E = 3200000
NUM_GRAPHS = 512
HIDDEN = 64

NP = 100352
EP = 3211264
RPS = NP // 16
ZROWS = 98
CHUNK = 512
PCH = CHUNK // 128
OUTER = (EP // 16) // CHUNK
OUTER_DEG = (EP // 32) // CHUNK
BLK = 2048
GRID = NP // BLK

_HI = lax.Precision.HIGHEST


def _mesh():
    return plsc.VectorSubcoreMesh(core_axis_name="c", subcore_axis_name="s")


_SC_PARAMS = pltpu.CompilerParams(use_tc_tiling_on_sc=False)


def _deg_body(dst_hbm, ones_hbm, zeros_hbm, out0, out1, dst_v, ones_v, deg_sh, ssem):
    c = lax.axis_index("c")
    s = lax.axis_index("s")
    pltpu.sync_copy(ones_hbm, ones_v)
    pltpu.sync_copy(zeros_hbm, deg_sh.at[pl.ds(s * RPS, RPS)])
    plsc.subcore_barrier()

    base = (c * 16 + s) * ((EP // 32) // CHUNK)

    def outer(i, carry):
        b = jnp.bitwise_and(i, 1)

        @pl.when(i >= 2)
        def _():
            pltpu.make_async_copy(ones_v, deg_sh.at[dst_v.at[b]],
                                  ssem).wait()

        @pl.when(i < OUTER_DEG)
        def _():
            pltpu.sync_copy(dst_hbm.at[base + i], dst_v.at[b])
            pltpu.async_copy(ones_v, deg_sh.at[dst_v.at[b]], ssem, add=True)
        return carry

    lax.fori_loop(0, OUTER_DEG + 2, outer, 0)
    plsc.subcore_barrier()
    r = pl.ds(s * RPS, RPS)

    @pl.when(c == 0)
    def _():
        pltpu.sync_copy(deg_sh.at[r], out0.at[r])

    @pl.when(c == 1)
    def _():
        pltpu.sync_copy(deg_sh.at[r], out1.at[r])


def _deg_call(dst3, ones_deg, zeros_deg):
    return pl.kernel(
        _deg_body,
        out_type=(jax.ShapeDtypeStruct((NP, 1), jnp.float32),
                  jax.ShapeDtypeStruct((NP, 1), jnp.float32)),
        mesh=_mesh(),
        scratch_types=[
            pltpu.VMEM((2, CHUNK), jnp.int32),
            pltpu.VMEM((CHUNK, 1), jnp.float32),
            pltpu.VMEM_SHARED((NP, 1), jnp.float32),
            pltpu.SemaphoreType.DMA,
        ],
        compiler_params=_SC_PARAMS,
    )(dst3, ones_deg, zeros_deg)


def _agg_body(hs0, hs1, hs2, hs3, ei_hbm, zeros_hbm,
              out0, out1, out2, out3,
              idx_v, rows_v, zbuf, agg_sh, gsem, ssem, isem):
    c = lax.axis_index("c")
    s = lax.axis_index("s")
    pltpu.sync_copy(zeros_hbm, zbuf)

    def do_quarter(hs_hbm, out_hbm):
        row0 = s * RPS
        for z in range(RPS // ZROWS):
            pltpu.sync_copy(zbuf, agg_sh.at[pl.ds(row0 + z * ZROWS, ZROWS)])
        plsc.subcore_barrier()

        base = s * ((EP // 16) // CHUNK)

        def fire_idx(i):
            pltpu.async_copy(ei_hbm.at[base + i], idx_v.at[lax.rem(i, 4)],
                             isem)

        def wait_idx(i):
            pltpu.make_async_copy(ei_hbm.at[base + i],
                                  idx_v.at[lax.rem(i, 4)], isem).wait()

        fire_idx(0)

        def outer(i, carry):
            r3 = lax.rem(i, 3)
            r4 = lax.rem(i, 4)

            @pl.when(jnp.logical_and(i >= 3, i <= OUTER + 2))
            def _():
                p4 = lax.rem(i - 3, 4)
                pltpu.make_async_copy(
                    rows_v.at[pl.ds(r3 * CHUNK, CHUNK)],
                    agg_sh.at[idx_v.at[p4, 1]], ssem).wait()

            @pl.when(i < OUTER)
            def _():
                wait_idx(i)

                @pl.when(i + 1 < OUTER)
                def _():
                    fire_idx(i + 1)

                pltpu.async_copy(hs_hbm.at[idx_v.at[r4, 0]],
                                 rows_v.at[pl.ds(r3 * CHUNK, CHUNK)], gsem)

            @pl.when(jnp.logical_and(i >= 1, i <= OUTER))
            def _():
                p3 = lax.rem(i - 1, 3)
                p4 = lax.rem(i - 1, 4)
                pltpu.make_async_copy(
                    hs_hbm.at[idx_v.at[p4, 0]],
                    rows_v.at[pl.ds(p3 * CHUNK, CHUNK)], gsem).wait()
                pltpu.async_copy(rows_v.at[pl.ds(p3 * CHUNK, CHUNK)],
                                 agg_sh.at[idx_v.at[p4, 1]], ssem, add=True)
            return carry

        lax.fori_loop(0, OUTER + 3, outer, 0)
        plsc.subcore_barrier()
        for z in range(RPS // ZROWS):
            r = pl.ds(row0 + z * ZROWS, ZROWS)
            pltpu.sync_copy(agg_sh.at[r], out_hbm.at[r])
        plsc.subcore_barrier()

    @pl.when(c == 0)
    def _():
        do_quarter(hs0, out0)
        do_quarter(hs1, out1)

    @pl.when(c == 1)
    def _():
        do_quarter(hs2, out2)
        do_quarter(hs3, out3)


def _agg_call(hsq, ei2, zeros_agg):
    q16 = jax.ShapeDtypeStruct((NP, 16), jnp.float32)
    return pl.kernel(
        _agg_body,
        out_type=(q16, q16, q16, q16),
        mesh=_mesh(),
        scratch_types=[
            pltpu.VMEM((4, 2, CHUNK), jnp.int32),
            pltpu.VMEM((3 * CHUNK, 16), jnp.float32),
            pltpu.VMEM((ZROWS, 16), jnp.float32),
            pltpu.VMEM_SHARED((NP, 16), jnp.float32),
            pltpu.SemaphoreType.DMA,
            pltpu.SemaphoreType.DMA,
            pltpu.SemaphoreType.DMA,
        ],
        compiler_params=_SC_PARAMS,
    )(hsq[0], hsq[1], hsq[2], hsq[3], ei2, zeros_agg)


_QSPEC = pl.BlockSpec((BLK, 16), lambda i: (i, 0))
_CSPEC = pl.BlockSpec((BLK, 1), lambda i: (i, 0))


def _k1_body(deg0_ref, deg1_ref, x_ref, w_ref,
             o0, o1, o2, o3, dis_ref):
    dis = lax.rsqrt(deg0_ref[...] + deg1_ref[...] + 1.0)
    h = jnp.dot(x_ref[...], w_ref[...], precision=_HI)
    for q, o in enumerate((o0, o1, o2, o3)):
        o[...] = dis * h[:, 16 * q:16 * (q + 1)]
    dis_ref[...] = dis


def _k1_call(deg0, deg1, xP, W1):
    q16 = jax.ShapeDtypeStruct((NP, 16), jnp.float32)
    outs = pl.pallas_call(
        _k1_body,
        grid=(GRID,),
        in_specs=[
            _CSPEC,
            _CSPEC,
            pl.BlockSpec((BLK, xP.shape[1]), lambda i: (i, 0)),
            pl.BlockSpec(W1.shape, lambda i: (0, 0)),
        ],
        out_specs=[_QSPEC, _QSPEC, _QSPEC, _QSPEC, _CSPEC],
        out_shape=[q16, q16, q16, q16,
                   jax.ShapeDtypeStruct((NP, 1), jnp.float32)],
    )(deg0, deg1, xP, W1)
    return outs[:4], outs[4]


def _k2_body(a0, a1, a2, a3, h0, h1, h2, h3, dis_ref, b_ref, w_ref,
             o0, o1, o2, o3):
    dis = dis_ref[...]
    aq = (a0, a1, a2, a3)
    hq = (h0, h1, h2, h3)
    h = None
    for q in range(4):
        xn = jnp.maximum(dis * (aq[q][...] + hq[q][...])
                         + b_ref[:, 16 * q:16 * (q + 1)], 0.0)
        p = jnp.dot(xn, w_ref[16 * q:16 * (q + 1), :], precision=_HI)
        h = p if h is None else h + p
    for q, o in enumerate((o0, o1, o2, o3)):
        o[...] = dis * h[:, 16 * q:16 * (q + 1)]


def _k2_call(aggq, hsq, dis, b_row, W):
    q16 = jax.ShapeDtypeStruct((NP, 16), jnp.float32)
    return pl.pallas_call(
        _k2_body,
        grid=(GRID,),
        in_specs=[
            _QSPEC, _QSPEC, _QSPEC, _QSPEC,
            _QSPEC, _QSPEC, _QSPEC, _QSPEC,
            _CSPEC,
            pl.BlockSpec((1, HIDDEN), lambda i: (0, 0)),
            pl.BlockSpec((HIDDEN, HIDDEN), lambda i: (0, 0)),
        ],
        out_specs=[_QSPEC, _QSPEC, _QSPEC, _QSPEC],
        out_shape=[q16, q16, q16, q16],
    )(*aggq, *hsq, dis, b_row, W)


def _k4_body(a0, a1, a2, a3, h0, h1, h2, h3, dis_ref, b_ref, batch_ref,
             wl_ref, bl_ref, out_ref, segsum, cnt):
    i = pl.program_id(0)

    @pl.when(i == 0)
    def _():
        segsum[...] = jnp.zeros_like(segsum)
        cnt[...] = jnp.zeros_like(cnt)

    dis = dis_ref[...]
    iota = lax.broadcasted_iota(jnp.int32, (BLK, NUM_GRAPHS), 1)
    onehot = (batch_ref[...] == iota).astype(jnp.float32)
    aq = (a0, a1, a2, a3)
    hq = (h0, h1, h2, h3)
    for q in range(4):
        x3q = dis * (aq[q][...] + hq[q][...]) + b_ref[:, 16 * q:16 * (q + 1)]
        segsum[:, 16 * q:16 * (q + 1)] += lax.dot_general(
            onehot, x3q, (((0,), (0,)), ((), ())))
    cnt[...] += lax.dot_general(onehot, jnp.ones((BLK, 1), jnp.float32),
                                (((0,), (0,)), ((), ())))

    @pl.when(i == GRID - 1)
    def _():
        pooled = segsum[...] / jnp.maximum(cnt[...], 1.0)
        out_ref[...] = jnp.dot(pooled, wl_ref[...], precision=_HI) + bl_ref[...]


def _k4_call(aggq, hsq, dis, b_row, batchP, Wl, bl_row):
    return pl.pallas_call(
        _k4_body,
        grid=(GRID,),
        in_specs=[
            _QSPEC, _QSPEC, _QSPEC, _QSPEC,
            _QSPEC, _QSPEC, _QSPEC, _QSPEC,
            _CSPEC,
            pl.BlockSpec((1, HIDDEN), lambda i: (0, 0)),
            pl.BlockSpec((BLK, 1), lambda i: (i, 0)),
            pl.BlockSpec(Wl.shape, lambda i: (0, 0)),
            pl.BlockSpec((1, Wl.shape[1]), lambda i: (0, 0)),
        ],
        out_specs=pl.BlockSpec((NUM_GRAPHS, Wl.shape[1]), lambda i: (0, 0)),
        out_shape=jax.ShapeDtypeStruct((NUM_GRAPHS, Wl.shape[1]), jnp.float32),
        scratch_shapes=[
            pltpu.VMEM((NUM_GRAPHS, HIDDEN), jnp.float32),
            pltpu.VMEM((NUM_GRAPHS, 1), jnp.float32),
        ],
    )(*aggq, *hsq, dis, b_row, batchP, Wl, bl_row)


def kernel(x, edge_index, batch, W1, b1, W2, b2, W3, b3, Wl, bl):
    f32 = jnp.float32
    src, dst = edge_index[0], edge_index[1]
    npad = EP - E
    pad_idx = (N + (jnp.arange(npad, dtype=jnp.int32) % (NP - N))).astype(
        jnp.int32)
    src2 = jnp.concatenate([src, pad_idx]).reshape(EP // CHUNK, CHUNK)
    dst2 = jnp.concatenate([dst, pad_idx]).reshape(EP // CHUNK, CHUNK)
    ei2 = jnp.stack([src2, dst2], axis=1)
    dst3 = dst2
    xP = jnp.pad(x, ((0, NP - N), (0, 0)))
    batchP = jnp.pad(batch, (0, NP - N),
                     constant_values=NUM_GRAPHS).reshape(NP, 1)
    zeros_agg = jnp.zeros((ZROWS, 16), f32)
    zeros_deg = jnp.zeros((RPS, 1), f32)
    ones_deg = jnp.ones((CHUNK, 1), f32)

    deg0, deg1 = _deg_call(dst3, ones_deg, zeros_deg)
    hs1q, dis = _k1_call(deg0, deg1, xP, W1)
    agg1q = _agg_call(hs1q, ei2, zeros_agg)
    hs2q = _k2_call(agg1q, hs1q, dis, b1[None, :], W2)
    agg2q = _agg_call(hs2q, ei2, zeros_agg)
    hs3q = _k2_call(agg2q, hs2q, dis, b2[None, :], W3)
    agg3q = _agg_call(hs3q, ei2, zeros_agg)
    return _k4_call(agg3q, hs3q, dis, b3[None, :], batchP, Wl, bl[None, :])
